# baseline (speedup 1.0000x reference)
import functools

import jax
import jax.numpy as jnp
from jax import lax
from jax.experimental import pallas as pl
from jax.experimental.pallas import tpu as pltpu
from jax.experimental.pallas import tpu_sc as plsc

_B, _S, _D = 4096, 50, 64
_N = _B * _S
_NW = 32
_G = 8
_W = _G * _S
_BPW = _B // _NW
_NWIN = _BPW // _G
_TCBB = 128


def _slice_body(i_ref, o_ref):
    o_ref[...] = i_ref[:, :, : _D]


def kernel(x, emb_weight):
    idx = x.reshape(_N).astype(jnp.int32)
    table = jnp.pad(emb_weight, ((0, 0), (0, 128 - _D)))

    @functools.partial(
        pl.kernel,
        out_type=jax.ShapeDtypeStruct((_B, _S, 128), emb_weight.dtype),
        mesh=plsc.VectorSubcoreMesh(core_axis_name="c", subcore_axis_name="s"),
        scratch_types=[
            pltpu.VMEM((_W,), jnp.int32),
            pltpu.VMEM((_W, 128), jnp.float32),
            pltpu.SemaphoreType.DMA,
            pltpu.SemaphoreType.DMA,
        ],
    )
    def gather_kernel(table_hbm, idx_hbm, out_hbm, idx_v, rows_v, gsem, wsem):
        wid = lax.axis_index("s") * 2 + lax.axis_index("c")
        batch0 = wid * _BPW

        @pl.loop(0, _NWIN)
        def _(w):
            b0 = batch0 + w * _G
            pltpu.sync_copy(idx_hbm.at[pl.ds(b0 * _S, _W)], idx_v)
            pltpu.async_copy(table_hbm.at[idx_v], rows_v, gsem).wait()
            writes = []
            for g in range(_G):
                writes.append(
                    pltpu.async_copy(
                        rows_v.at[pl.ds(g * _S, _S)],
                        out_hbm.at[b0 + g],
                        wsem,
                    )
                )
            for h in writes:
                h.wait()

    wide = gather_kernel(table, idx)

    return pl.pallas_call(
        _slice_body,
        grid=(_B // _TCBB,),
        in_specs=[
            pl.BlockSpec((_TCBB, _S, 128), lambda i: (i, 0, 0)),
        ],
        out_specs=pl.BlockSpec((_TCBB, _S, _D), lambda i: (i, 0, 0)),
        out_shape=jax.ShapeDtypeStruct((_B, _S, _D), emb_weight.dtype),
    )(wide)


# speedup vs baseline: 1.2695x; 1.2695x over previous
import functools

import jax
import jax.numpy as jnp
from jax import lax
from jax.experimental import pallas as pl
from jax.experimental.pallas import tpu as pltpu
from jax.experimental.pallas import tpu_sc as plsc

_B, _S, _D = 4096, 50, 64
_N = _B * _S
_NW = 32
_G = 8
_W = _G * _S
_BPW = _B // _NW
_NWIN = _BPW // _G
_TCBB = 128


def _slice_body(i_ref, o_ref):
    o_ref[...] = jnp.transpose(i_ref[:, :, : _D], (1, 2, 0))


def kernel(x, emb_weight):
    idx = x.reshape(_N).astype(jnp.int32)
    table = jnp.pad(emb_weight, ((0, 0), (0, 128 - _D)))

    @functools.partial(
        pl.kernel,
        out_type=jax.ShapeDtypeStruct((_B, _S, 128), emb_weight.dtype),
        mesh=plsc.VectorSubcoreMesh(core_axis_name="c", subcore_axis_name="s"),
        scratch_types=[
            pltpu.VMEM((_W,), jnp.int32),
            pltpu.VMEM((_W, 128), jnp.float32),
            pltpu.SemaphoreType.DMA,
            pltpu.SemaphoreType.DMA,
        ],
    )
    def gather_kernel(table_hbm, idx_hbm, out_hbm, idx_v, rows_v, gsem, wsem):
        wid = lax.axis_index("s") * 2 + lax.axis_index("c")
        batch0 = wid * _BPW

        @pl.loop(0, _NWIN)
        def _(w):
            b0 = batch0 + w * _G
            pltpu.sync_copy(idx_hbm.at[pl.ds(b0 * _S, _W)], idx_v)
            pltpu.async_copy(table_hbm.at[idx_v], rows_v, gsem).wait()
            writes = []
            for g in range(_G):
                writes.append(
                    pltpu.async_copy(
                        rows_v.at[pl.ds(g * _S, _S)],
                        out_hbm.at[b0 + g],
                        wsem,
                    )
                )
            for h in writes:
                h.wait()

    wide = gather_kernel(table, idx)

    swapped = pl.pallas_call(
        _slice_body,
        grid=(_B // _TCBB,),
        in_specs=[
            pl.BlockSpec((_TCBB, _S, 128), lambda i: (i, 0, 0)),
        ],
        out_specs=pl.BlockSpec((_S, _D, _TCBB), lambda i: (0, 0, i)),
        out_shape=jax.ShapeDtypeStruct((_S, _D, _B), emb_weight.dtype),
    )(wide)
    return swapped.transpose(2, 0, 1)


# speedup vs baseline: 1.7083x; 1.3456x over previous
import functools

import jax
import jax.numpy as jnp
from jax.experimental import pallas as pl
from jax.experimental.pallas import tpu as pltpu
from jax.experimental.pallas import tpu_sc as plsc

_B, _S, _D = 4096, 50, 64
_N = _B * _S
_V = 100001
_W = 400
_TT = 8192
_SB = 5


def _prep_body(i_ref, o_ref):
    t = jnp.transpose(i_ref[...], (1, 0))
    o_ref[...] = jnp.concatenate([t, t], axis=1)


def _swap_body(i_ref, o_ref):
    o_ref[...] = jnp.transpose(i_ref[:, :, : _D], (0, 2, 1))


def kernel(x, emb_weight):
    idx = x.T.reshape(_N).astype(jnp.int32)

    table = pl.pallas_call(
        _prep_body,
        grid=(pl.cdiv(_V, _TT),),
        in_specs=[pl.BlockSpec((_D, _TT), lambda i: (0, i))],
        out_specs=pl.BlockSpec((_TT, 128), lambda i: (i, 0)),
        out_shape=jax.ShapeDtypeStruct((_V, 128), emb_weight.dtype),
    )(emb_weight.T)

    @functools.partial(
        pl.kernel,
        out_type=jax.ShapeDtypeStruct((_N, 128), emb_weight.dtype),
        mesh=plsc.VectorSubcoreMesh(core_axis_name="c", subcore_axis_name="s"),
    )
    def gather_kernel(table_hbm, idx_hbm, out_hbm):
        def body(idx_vmem, out_vmem):
            pltpu.sync_copy(table_hbm.at[idx_vmem], out_vmem)

        pltpu.emit_pipeline(
            body,
            grid=(_N // _W,),
            in_specs=[pl.BlockSpec((_W,), index_map=lambda i: (i,))],
            out_specs=[pl.BlockSpec((_W, 128), index_map=lambda i: (i, 0))],
            core_axis_name=("c", "s"),
            dimension_semantics=(pltpu.PARALLEL,),
        )(idx_hbm, out_hbm)

    wide = gather_kernel(table, idx).reshape(_S, _B, 128)

    swapped = pl.pallas_call(
        _swap_body,
        grid=(_S // _SB,),
        in_specs=[pl.BlockSpec((_SB, _B, 128), lambda i: (i, 0, 0))],
        out_specs=pl.BlockSpec((_SB, _D, _B), lambda i: (i, 0, 0)),
        out_shape=jax.ShapeDtypeStruct((_S, _D, _B), emb_weight.dtype),
    )(wide)

    return swapped.transpose(2, 0, 1)
